# Initial kernel scaffold; baseline (speedup 1.0000x reference)
import jax
import jax.numpy as jnp
from jax.experimental import pallas as pl


def kernel(x, adj, W1, W2, W3, Wsemi):
    raise NotImplementedError("write your pallas kernel here")


# speedup vs baseline: 1.2208x; 1.2208x over previous
import functools

import jax
import jax.numpy as jnp
from jax.experimental import pallas as pl
from jax.experimental.pallas import tpu as pltpu

N = 4096
D = 256
H1 = 64
H2 = 32
K = 16
HC = 2 * H2 + K

BM = 512
NB = N // BM


def _gcn_body(adj_ref, x_ref, w1_ref, wcat_ref,
              mean_ref, logvar_ref, z_ref, semi_ref, logsm_ref, sm_ref,
              p_ref, h1_ref, q_ref):
    s = pl.program_id(0)
    j = pl.program_id(1)

    @pl.when((s == 0) & (j == 0))
    def _():
        p_ref[...] = jnp.dot(x_ref[...], w1_ref[...],
                             preferred_element_type=jnp.float32)

    @pl.when(s == 0)
    def _():
        h1_ref[pl.ds(j * BM, BM), :] = jnp.maximum(
            jnp.dot(adj_ref[...], p_ref[...],
                    preferred_element_type=jnp.float32), 0.0)

    @pl.when((s == 1) & (j == 0))
    def _():
        q_ref[...] = jnp.dot(h1_ref[...], wcat_ref[...],
                             preferred_element_type=jnp.float32)

    @pl.when(s == 1)
    def _():
        out2 = jnp.dot(adj_ref[...], q_ref[...],
                       preferred_element_type=jnp.float32)
        mean = out2[:, :H2]
        logvar = out2[:, H2:2 * H2]
        semi = out2[:, 2 * H2:]
        mean_ref[...] = mean
        z_ref[...] = mean
        logvar_ref[...] = logvar
        semi_ref[...] = semi
        m = jnp.max(semi, axis=1, keepdims=True)
        shifted = semi - m
        e = jnp.exp(shifted)
        ssum = jnp.sum(e, axis=1, keepdims=True)
        sm_ref[...] = e / ssum
        logsm_ref[...] = shifted - jnp.log(ssum)


def _recon_body(zb_ref, zall_ref, out_ref):
    out_ref[...] = jax.lax.dot_general(
        zb_ref[...], zall_ref[...],
        dimension_numbers=(((1,), (1,)), ((), ())),
        preferred_element_type=jnp.float32)


@jax.jit
def kernel(x, adj, W1, W2, W3, Wsemi):
    wcat = jnp.concatenate([W2, W3, Wsemi], axis=1)

    small = lambda w: (N, w)
    mean, logvar, z, semi, logsm, sm = pl.pallas_call(
        _gcn_body,
        grid=(2, NB),
        in_specs=[
            pl.BlockSpec((BM, N), lambda s, j: (j, 0)),
            pl.BlockSpec((N, D), lambda s, j: (0, 0)),
            pl.BlockSpec((D, H1), lambda s, j: (0, 0)),
            pl.BlockSpec((H1, HC), lambda s, j: (0, 0)),
        ],
        out_specs=[
            pl.BlockSpec((BM, H2), lambda s, j: (j, 0)),
            pl.BlockSpec((BM, H2), lambda s, j: (j, 0)),
            pl.BlockSpec((BM, H2), lambda s, j: (j, 0)),
            pl.BlockSpec((BM, K), lambda s, j: (j, 0)),
            pl.BlockSpec((BM, K), lambda s, j: (j, 0)),
            pl.BlockSpec((BM, K), lambda s, j: (j, 0)),
        ],
        out_shape=[
            jax.ShapeDtypeStruct(small(H2), jnp.float32),
            jax.ShapeDtypeStruct(small(H2), jnp.float32),
            jax.ShapeDtypeStruct(small(H2), jnp.float32),
            jax.ShapeDtypeStruct(small(K), jnp.float32),
            jax.ShapeDtypeStruct(small(K), jnp.float32),
            jax.ShapeDtypeStruct(small(K), jnp.float32),
        ],
        scratch_shapes=[
            pltpu.VMEM((N, H1), jnp.float32),
            pltpu.VMEM((N, H1), jnp.float32),
            pltpu.VMEM((N, HC), jnp.float32),
        ],
        compiler_params=pltpu.CompilerParams(
            dimension_semantics=("arbitrary", "arbitrary")),
    )(adj, x, W1, wcat)

    adj_recon = pl.pallas_call(
        _recon_body,
        grid=(NB,),
        in_specs=[
            pl.BlockSpec((BM, H2), lambda j: (j, 0)),
            pl.BlockSpec((N, H2), lambda j: (0, 0)),
        ],
        out_specs=pl.BlockSpec((BM, N), lambda j: (j, 0)),
        out_shape=jax.ShapeDtypeStruct((N, N), jnp.float32),
        compiler_params=pltpu.CompilerParams(
            dimension_semantics=("arbitrary",)),
    )(z, z)

    return (adj_recon, mean, logvar, z, logsm, sm, semi)
